# baseline (speedup 1.0000x reference)
import jax
import jax.numpy as jnp
from jax import lax
from jax.experimental import pallas as pl
from jax.experimental.pallas import tpu as pltpu
from jax.experimental.pallas import tpu_sc as plsc

NCLS = 18048
NFEAT = 256
NBATCH = 4096

NW = 32
BPW = NBATCH // NW
RCHUNK = 128
NCHUNK = NCLS // RCHUNK
KMAX = -(-NCHUNK // NW)

N_BLK = 384
MM_STEPS = NCLS // N_BLK
U_STEPS = (NBATCH + N_BLK - 1) // N_BLK
LO_BLK = 128
LO_STEPS = NBATCH // LO_BLK


def _sc_mesh():
    return plsc.VectorSubcoreMesh(core_axis_name="c", subcore_axis_name="s")


def _wid():
    return lax.axis_index("s") * 2 + lax.axis_index("c")


def _gather_body(mem_hbm, tgt_hbm, g_hbm, nm0_hbm, idx_v, rows_v, cp_v, sem):
    w = _wid()
    base = w * BPW
    pltpu.sync_copy(tgt_hbm.at[pl.ds(base, BPW)], idx_v)
    pltpu.async_copy(mem_hbm.at[idx_v], rows_v, sem).wait()
    pltpu.sync_copy(rows_v, g_hbm.at[pl.ds(base, BPW)])

    for k in range(KMAX):
        c = w + k * NW

        @pl.when(c < NCHUNK)
        def _copy_chunk():
            b2 = c * RCHUNK
            pltpu.sync_copy(mem_hbm.at[pl.ds(b2, RCHUNK)], cp_v)
            pltpu.sync_copy(cp_v, nm0_hbm.at[pl.ds(b2, RCHUNK)])


def _sc_gather(mem, targets):
    k = pl.kernel(
        _gather_body,
        out_type=(
            jax.ShapeDtypeStruct((NBATCH, NFEAT), jnp.float32),
            jax.ShapeDtypeStruct((NCLS, NFEAT), jnp.float32),
        ),
        mesh=_sc_mesh(),
        scratch_types=[
            pltpu.VMEM((BPW,), jnp.int32),
            pltpu.VMEM((BPW, NFEAT), jnp.float32),
            pltpu.VMEM((RCHUNK, NFEAT), jnp.float32),
            pltpu.SemaphoreType.DMA,
        ],
        name="sc_gather_copy",
    )
    return k(mem, targets)


def _mm_body(x_mm_ref, mem_ref, t_col_ref, t_row_ref, logits_ref, lo_ref):
    s = pl.program_id(0)

    logits_ref[...] = lax.dot_general(
        x_mm_ref[...], mem_ref[...], (((1,), (1,)), ((), ())),
        preferred_element_type=jnp.float32)

    b = lax.rem(s, LO_STEPS)
    tb = t_col_ref[pl.ds(b * LO_BLK, LO_BLK), :]
    eq = tb == t_row_ref[...]
    jidx = lax.broadcasted_iota(jnp.int32, (LO_BLK, NBATCH), 1)
    lo_ref[pl.ds(b * LO_BLK, LO_BLK), :] = jnp.max(
        jnp.where(eq, jidx, -1), axis=1, keepdims=True)


def _tc_main(inputs, mem, t_col, t_row):
    return pl.pallas_call(
        _mm_body,
        grid=(MM_STEPS,),
        in_specs=[
            pl.BlockSpec((NBATCH, NFEAT), lambda s: (0, 0)),
            pl.BlockSpec((N_BLK, NFEAT), lambda s: (s, 0)),
            pl.BlockSpec((NBATCH, 1), lambda s: (0, 0)),
            pl.BlockSpec((1, NBATCH), lambda s: (0, 0)),
        ],
        out_specs=[
            pl.BlockSpec((NBATCH, N_BLK), lambda s: (0, s)),
            pl.BlockSpec((NBATCH, 1), lambda s: (0, 0)),
        ],
        out_shape=[
            jax.ShapeDtypeStruct((NBATCH, NCLS), jnp.float32),
            jax.ShapeDtypeStruct((NBATCH, 1), jnp.int32),
        ],
        name="tc_matmul_lo",
    )(inputs, mem, t_col, t_row)


def _upd_body(alpha_ref, g_ref, x_ref, u_ref):
    a = alpha_ref[0, 0]
    u = a * g_ref[...] + (1.0 - a) * x_ref[...]
    n = jnp.sqrt(jnp.sum(u * u, axis=1, keepdims=True))
    u_ref[...] = u / (n + 1e-12)


def _tc_update(alpha, g, inputs):
    return pl.pallas_call(
        _upd_body,
        in_specs=[
            pl.BlockSpec(memory_space=pltpu.SMEM),
            pl.BlockSpec((NBATCH, NFEAT), lambda: (0, 0)),
            pl.BlockSpec((NBATCH, NFEAT), lambda: (0, 0)),
        ],
        out_specs=pl.BlockSpec((NBATCH, NFEAT), lambda: (0, 0)),
        out_shape=jax.ShapeDtypeStruct((NBATCH, NFEAT), jnp.float32),
        name="tc_update_u",
    )(alpha, g, inputs)


def _scatterf_body(u_hbm, tgt_hbm, lo_hbm, nm_hbm, idx_v, lo_v, rows_v, sem):
    w = _wid()
    base = w * BPW
    pltpu.sync_copy(tgt_hbm.at[pl.ds(base, BPW)], idx_v)
    pltpu.sync_copy(lo_hbm.at[pl.ds(base, BPW)], lo_v)
    pltpu.async_copy(u_hbm.at[lo_v], rows_v, sem).wait()
    pltpu.async_copy(rows_v, nm_hbm.at[idx_v], sem).wait()


def _sc_scatterf(u, targets, lastocc, nm_ref):
    k = pl.kernel(
        _scatterf_body,
        out_type=(),
        mesh=_sc_mesh(),
        scratch_types=[
            pltpu.VMEM((BPW,), jnp.int32),
            pltpu.VMEM((BPW,), jnp.int32),
            pltpu.VMEM((BPW, NFEAT), jnp.float32),
            pltpu.SemaphoreType.DMA,
        ],
        name="sc_scatter_winners",
    )
    k(u, targets, lastocc, nm_ref)


def kernel(inputs, targets, mem, epoch):
    t32 = targets.astype(jnp.int32)
    alpha = jnp.asarray(0.5 * epoch / 60.0, jnp.float32).reshape(1, 1)

    g, nm0 = _sc_gather(mem, t32)
    logits, lastocc = _tc_main(
        inputs, mem, t32.reshape(NBATCH, 1), t32.reshape(1, NBATCH))
    u = _tc_update(alpha, g, inputs)
    nm_ref = jax.new_ref(nm0)
    _sc_scatterf(u, t32, lastocc.reshape(NBATCH), nm_ref)
    return logits, nm_ref[...]


# speedup vs baseline: 1.0745x; 1.0745x over previous
import jax
import jax.numpy as jnp
from jax import lax
from jax.experimental import pallas as pl
from jax.experimental.pallas import tpu as pltpu
from jax.experimental.pallas import tpu_sc as plsc

NCLS = 18048
NFEAT = 256
NBATCH = 4096

NW = 32
BPW = NBATCH // NW

N_BLK = 384
MM_STEPS = NCLS // N_BLK
LO_BLK = 128
LO_STEPS = NBATCH // LO_BLK


def _sc_mesh():
    return plsc.VectorSubcoreMesh(core_axis_name="c", subcore_axis_name="s")


def _wid():
    return lax.axis_index("s") * 2 + lax.axis_index("c")


def _gather_body(mem_hbm, tgt_hbm, out_hbm, idx_v, rows_v, sem):
    w = _wid()
    base = w * BPW
    pltpu.sync_copy(tgt_hbm.at[pl.ds(base, BPW)], idx_v)
    pltpu.async_copy(mem_hbm.at[idx_v], rows_v, sem).wait()
    pltpu.sync_copy(rows_v, out_hbm.at[pl.ds(base, BPW)])


def _sc_gather(mem, targets):
    k = pl.kernel(
        _gather_body,
        out_type=jax.ShapeDtypeStruct((NBATCH, NFEAT), jnp.float32),
        mesh=_sc_mesh(),
        scratch_types=[
            pltpu.VMEM((BPW,), jnp.int32),
            pltpu.VMEM((BPW, NFEAT), jnp.float32),
            pltpu.SemaphoreType.DMA,
        ],
        name="sc_gather_rows",
    )
    return k(mem, targets)


def _mm_body(x_mm_ref, mem_ref, t_col_ref, t_row_ref,
             logits_ref, nm0_ref, lo_ref):
    s = pl.program_id(0)

    m = mem_ref[...]
    logits_ref[...] = lax.dot_general(
        x_mm_ref[...], m, (((1,), (1,)), ((), ())),
        preferred_element_type=jnp.float32)
    nm0_ref[...] = m

    b = lax.rem(s, LO_STEPS)
    tb = t_col_ref[pl.ds(b * LO_BLK, LO_BLK), :]
    eq = tb == t_row_ref[...]
    jidx = lax.broadcasted_iota(jnp.int32, (LO_BLK, NBATCH), 1)
    lo_ref[pl.ds(b * LO_BLK, LO_BLK), :] = jnp.max(
        jnp.where(eq, jidx, -1), axis=1, keepdims=True)


def _tc_main(inputs, mem, t_col, t_row):
    return pl.pallas_call(
        _mm_body,
        grid=(MM_STEPS,),
        in_specs=[
            pl.BlockSpec((NBATCH, NFEAT), lambda s: (0, 0)),
            pl.BlockSpec((N_BLK, NFEAT), lambda s: (s, 0)),
            pl.BlockSpec((NBATCH, 1), lambda s: (0, 0)),
            pl.BlockSpec((1, NBATCH), lambda s: (0, 0)),
        ],
        out_specs=[
            pl.BlockSpec((NBATCH, N_BLK), lambda s: (0, s)),
            pl.BlockSpec((N_BLK, NFEAT), lambda s: (s, 0)),
            pl.BlockSpec((NBATCH, 1), lambda s: (0, 0)),
        ],
        out_shape=[
            jax.ShapeDtypeStruct((NBATCH, NCLS), jnp.float32),
            jax.ShapeDtypeStruct((NCLS, NFEAT), jnp.float32),
            jax.ShapeDtypeStruct((NBATCH, 1), jnp.int32),
        ],
        name="tc_matmul_lo_nm",
    )(inputs, mem, t_col, t_row)


def _upd_body(alpha_ref, g_ref, x_ref, u_ref):
    a = alpha_ref[0, 0]
    u = a * g_ref[...] + (1.0 - a) * x_ref[...]
    n = jnp.sqrt(jnp.sum(u * u, axis=1, keepdims=True))
    u_ref[...] = u / (n + 1e-12)


def _tc_update(alpha, g, inputs):
    return pl.pallas_call(
        _upd_body,
        in_specs=[
            pl.BlockSpec(memory_space=pltpu.SMEM),
            pl.BlockSpec((NBATCH, NFEAT), lambda: (0, 0)),
            pl.BlockSpec((NBATCH, NFEAT), lambda: (0, 0)),
        ],
        out_specs=pl.BlockSpec((NBATCH, NFEAT), lambda: (0, 0)),
        out_shape=jax.ShapeDtypeStruct((NBATCH, NFEAT), jnp.float32),
        name="tc_update_u",
    )(alpha, g, inputs)


def _scatterf_body(u_hbm, tgt_hbm, lo_hbm, nm_hbm, idx_v, lo_v, rows_v, sem):
    w = _wid()
    base = w * BPW
    pltpu.sync_copy(tgt_hbm.at[pl.ds(base, BPW)], idx_v)
    pltpu.sync_copy(lo_hbm.at[pl.ds(base, BPW)], lo_v)
    pltpu.async_copy(u_hbm.at[lo_v], rows_v, sem).wait()
    pltpu.async_copy(rows_v, nm_hbm.at[idx_v], sem).wait()


def _sc_scatterf(u, targets, lastocc, nm_ref):
    k = pl.kernel(
        _scatterf_body,
        out_type=(),
        mesh=_sc_mesh(),
        scratch_types=[
            pltpu.VMEM((BPW,), jnp.int32),
            pltpu.VMEM((BPW,), jnp.int32),
            pltpu.VMEM((BPW, NFEAT), jnp.float32),
            pltpu.SemaphoreType.DMA,
        ],
        name="sc_scatter_winners",
    )
    k(u, targets, lastocc, nm_ref)


def kernel(inputs, targets, mem, epoch):
    t32 = targets.astype(jnp.int32)
    alpha = jnp.asarray(0.5 * epoch / 60.0, jnp.float32).reshape(1, 1)

    g = _sc_gather(mem, t32)
    logits, nm0, lastocc = _tc_main(
        inputs, mem, t32.reshape(NBATCH, 1), t32.reshape(1, NBATCH))
    u = _tc_update(alpha, g, inputs)
    nm_ref = jax.new_ref(nm0)
    _sc_scatterf(u, t32, lastocc.reshape(NBATCH), nm_ref)
    return logits, nm_ref[...]
